# Initial kernel scaffold; baseline (speedup 1.0000x reference)
import jax
import jax.numpy as jnp
from jax.experimental import pallas as pl


def kernel(inputs, targets):
    raise NotImplementedError("write your pallas kernel here")


# speedup vs baseline: 3.4883x; 3.4883x over previous
import jax
import jax.numpy as jnp
from jax.experimental import pallas as pl
from jax.experimental.pallas import tpu as pltpu

IGNORE_INDEX = 255
BOUNDARY_WEIGHT = 2.0

B, C, H, W = 8, 21, 512, 512
BH = 16
GRID = H // BH


def _loss_kernel(x_ref, tup_ref, tmd_ref, tdn_ref, out_ref):
    i = pl.program_id(0)

    tup = tup_ref[...]
    tmd = tmd_ref[...]
    tdn = tdn_ref[...]

    rmax = jnp.maximum(jnp.maximum(tup, tmd), tdn)
    rmin = jnp.minimum(jnp.minimum(tup, tmd), tdn)
    diff = (rmax - rmin) > 0
    cany = diff[:, :, 0:W] | diff[:, :, 1 : W + 1] | diff[:, :, 2 : W + 2]
    bmap = jnp.any(cany, axis=0)

    ri = jax.lax.broadcasted_iota(jnp.int32, (BH, W), 0) + i * BH
    ci = jax.lax.broadcasted_iota(jnp.int32, (BH, W), 1)
    interior = (ri >= 1) & (ri <= H - 2) & (ci >= 1) & (ci <= W - 2)
    wgt = jnp.where(bmap & interior, 1.0 + BOUNDARY_WEIGHT, 1.0)

    t = tmd[:, :, 1 : W + 1]
    x = x_ref[...]
    m = jnp.max(x, axis=1)
    e = jnp.exp(x - m[:, None, :, :])
    s = jnp.sum(e, axis=1)
    cidx = jax.lax.broadcasted_iota(jnp.int32, (B, C, BH, W), 1)
    xt = jnp.sum(jnp.where(cidx == t[:, None, :, :], x, 0.0), axis=1)
    ce = m + jnp.log(s) - xt
    ce = jnp.where(t != IGNORE_INDEX, ce, 0.0)

    contrib = ce * wgt[None, :, :]
    red = jnp.sum(contrib, axis=0)
    folded = (
        red[:, 0:128] + red[:, 128:256] + red[:, 256:384] + red[:, 384:512]
    )
    out_ref[0, :, :] = folded


def kernel(inputs, targets):
    t32 = targets.astype(jnp.int32)
    tp = jnp.pad(t32, ((0, 0), (1, 1), (1, 1)), mode="edge")
    tup = tp[:, 0:H, :]
    tmd = tp[:, 1 : H + 1, :]
    tdn = tp[:, 2 : H + 2, :]

    partials = pl.pallas_call(
        _loss_kernel,
        grid=(GRID,),
        in_specs=[
            pl.BlockSpec((B, C, BH, W), lambda i: (0, 0, i, 0)),
            pl.BlockSpec((B, BH, W + 2), lambda i: (0, i, 0)),
            pl.BlockSpec((B, BH, W + 2), lambda i: (0, i, 0)),
            pl.BlockSpec((B, BH, W + 2), lambda i: (0, i, 0)),
        ],
        out_specs=pl.BlockSpec((1, BH, 128), lambda i: (i, 0, 0)),
        out_shape=jax.ShapeDtypeStruct((GRID, BH, 128), jnp.float32),
        compiler_params=pltpu.CompilerParams(
            dimension_semantics=("parallel",),
        ),
    )(inputs, tup, tmd, tdn)

    return jnp.sum(partials) / jnp.float32(B * H * W)
